# baseline (speedup 1.0000x reference)
import jax
import jax.numpy as jnp
from jax.experimental import pallas as pl
from jax.experimental.pallas import tpu as pltpu

_BM = 400
_SUB = 2
_BR = _BM // _SUB


def _gcn_block(seq_ref, adj_ref, w_ref, a_ref, out_ref):
    r = pl.program_id(0) % _SUB
    seqb = seq_ref[...].astype(jnp.bfloat16)

    def run(lo):
        h = jnp.dot(adj_ref[lo:lo + _BR, :].astype(jnp.bfloat16), seqb,
                    preferred_element_type=jnp.float32)
        y = jax.lax.dot_general(h, w_ref[...], (((1,), (1,)), ((), ())),
                                preferred_element_type=jnp.float32)
        slope = a_ref[0, 0]
        out_ref[...] = jnp.where(y >= 0, y, slope * y)

    for s in range(_SUB):
        @pl.when(r == s)
        def _(s=s):
            run(s * _BR)


def kernel(seq, adj, W, a):
    N, d_in = seq.shape
    d_out = W.shape[0]
    return pl.pallas_call(
        _gcn_block,
        grid=(N // _BR,),
        in_specs=[
            pl.BlockSpec((N, d_in), lambda i: (0, 0)),
            pl.BlockSpec((_BM, N), lambda i: (i // _SUB, 0)),
            pl.BlockSpec((d_out, d_in), lambda i: (0, 0)),
            pl.BlockSpec(memory_space=pltpu.SMEM),
        ],
        out_specs=pl.BlockSpec((_BR, d_out), lambda i: (i, 0)),
        out_shape=jax.ShapeDtypeStruct((N, d_out), jnp.float32),
    )(seq, adj, W, a.reshape(1, 1))


# speedup vs baseline: 1.5118x; 1.5118x over previous
import jax
import jax.numpy as jnp
from jax.experimental import pallas as pl
from jax.experimental.pallas import tpu as pltpu

_BM = 400


def _gcn_block(seq_ref, adj_ref, w_ref, a_ref, out_ref, seqb_ref):
    @pl.when(pl.program_id(0) == 0)
    def _cache_seq():
        seqb_ref[...] = seq_ref[...].astype(jnp.bfloat16)

    h = jnp.dot(adj_ref[...].astype(jnp.bfloat16), seqb_ref[...],
                preferred_element_type=jnp.float32)
    y = jax.lax.dot_general(h, w_ref[...], (((1,), (1,)), ((), ())),
                            preferred_element_type=jnp.float32)
    slope = a_ref[0, 0]
    out_ref[...] = jnp.where(y >= 0, y, slope * y)


def kernel(seq, adj, W, a):
    N, d_in = seq.shape
    d_out = W.shape[0]
    return pl.pallas_call(
        _gcn_block,
        grid=(N // _BM,),
        in_specs=[
            pl.BlockSpec((N, d_in), lambda i: (0, 0)),
            pl.BlockSpec((_BM, N), lambda i: (i, 0)),
            pl.BlockSpec((d_out, d_in), lambda i: (0, 0)),
            pl.BlockSpec(memory_space=pltpu.SMEM),
        ],
        out_specs=pl.BlockSpec((_BM, d_out), lambda i: (i, 0)),
        out_shape=jax.ShapeDtypeStruct((N, d_out), jnp.float32),
        scratch_shapes=[pltpu.VMEM((N, d_in), jnp.bfloat16)],
    )(seq, adj, W, a.reshape(1, 1))
